# baseline (speedup 1.0000x reference)
import functools

import jax
import jax.numpy as jnp
from jax import lax
from jax.experimental import pallas as pl
from jax.experimental.pallas import tpu as pltpu
from jax.experimental.pallas import tpu_sc as plsc

N = 50000
E = 800000
G = 128

NC = 2
NS = 16
NW = NC * NS
CHUNK = 128
CPB = 14
NBLK = 14
CPW = CPB * NBLK
EPW = CPW * CHUNK
EP = EPW * NW
RPT = 1568
R = RPT * NW
OUTC = 224


def _sc_aggregate(F, scale_all):
    mesh = plsc.VectorSubcoreMesh(core_axis_name="c", subcore_axis_name="s")

    @functools.partial(
        pl.kernel,
        mesh=mesh,
        out_type=jax.ShapeDtypeStruct((NC * R, F), jnp.float32),
        compiler_params=pltpu.CompilerParams(use_tc_tiling_on_sc=False),
        scratch_types=[
            pltpu.VMEM((CPB, CHUNK), jnp.int32),
            pltpu.VMEM((CPB, CHUNK), jnp.int32),
            pltpu.VMEM((CPB * CHUNK,), jnp.float32),
            pltpu.VMEM((CHUNK, F), jnp.float32),
            pltpu.VMEM((CHUNK, F), jnp.float32),
            pltpu.VMEM_SHARED((R, F), jnp.float32),
            pltpu.SemaphoreType.DMA,
            pltpu.SemaphoreType.DMA,
            pltpu.SemaphoreType.DMA,
            pltpu.SemaphoreType.DMA,
        ],
    )
    def agg(table, srcT, dstT, ewT, zrows, out, src_v, dst_v, ew_v, rows_v,
            rows_w, acc_sh, sem, sem2, sem3, sem4):
        c = lax.axis_index("c")
        s = lax.axis_index("s")
        row0 = s * RPT

        pltpu.sync_copy(zrows.at[pl.ds(row0, RPT)],
                        acc_sh.at[pl.ds(row0, RPT)])
        plsc.subcore_barrier()

        lane4 = lax.iota(jnp.int32, 16) < 4

        def scale(buf, ch):
            for k in range(CHUNK // 16):
                wvec = ew_v[pl.ds(ch * CHUNK + 16 * k, 16)]
                for jj in range(16):
                    j = 16 * k + jj
                    w = wvec[jj]
                    if scale_all:
                        for h in range(F // 16):
                            sl = pl.ds(16 * h, 16)
                            buf[j, sl] = buf[j, sl] * w
                    else:
                        sl = pl.ds(0, 16)
                        m = jnp.where(lane4, w, 1.0)
                        buf[j, sl] = buf[j, sl] * m

        def block(b, _):
            pltpu.sync_copy(srcT.at[c, s, b], src_v)
            pltpu.sync_copy(dstT.at[c, s, b], dst_v)
            pltpu.sync_copy(ewT.at[c, s, b], ew_v)
            pltpu.async_copy(table.at[src_v.at[0]], rows_v, sem)

            def pair(p, _):
                ch0 = 2 * p
                pltpu.make_async_copy(table.at[src_v.at[ch0]], rows_v,
                                      sem).wait()

                @pl.when(p > 0)
                def _drain_w():
                    pltpu.make_async_copy(rows_w, acc_sh.at[dst_v.at[ch0]],
                                          sem4).wait()
                pltpu.async_copy(table.at[src_v.at[ch0 + 1]], rows_w, sem2)
                scale(rows_v, ch0)
                pltpu.async_copy(rows_v, acc_sh.at[dst_v.at[ch0]], sem3,
                                 add=True)
                pltpu.make_async_copy(table.at[src_v.at[ch0 + 1]], rows_w,
                                      sem2).wait()
                pltpu.make_async_copy(rows_v, acc_sh.at[dst_v.at[ch0]],
                                      sem3).wait()

                @pl.when(p < CPB // 2 - 1)
                def _start_next():
                    pltpu.async_copy(table.at[src_v.at[ch0 + 2]], rows_v,
                                     sem)
                scale(rows_w, ch0 + 1)
                pltpu.async_copy(rows_w, acc_sh.at[dst_v.at[ch0 + 1]], sem4,
                                 add=True)
                return _
            lax.fori_loop(0, CPB // 2, pair, 0)
            pltpu.make_async_copy(rows_w, acc_sh.at[dst_v.at[CPB - 1]],
                                  sem4).wait()
            return _
        lax.fori_loop(0, NBLK, block, 0)
        plsc.subcore_barrier()

        pltpu.sync_copy(acc_sh.at[pl.ds(row0, RPT)],
                        out.at[pl.ds(c * R + row0, RPT)])

    return agg


_agg1 = _sc_aggregate(16, False)
_agg2 = _sc_aggregate(16, True)
_agg3 = _sc_aggregate(32, True)


BLK = RPT
_TCGRID = R // BLK


def _full(shape):
    return pl.BlockSpec(shape, lambda *_: tuple(0 for _ in shape))


def _rows(w):
    return pl.BlockSpec((BLK, w), lambda i: (i, 0))


def _rows2(w, off):
    return pl.BlockSpec((BLK, w), lambda i: (off + i, 0))


def _tc1_body(a0, a1, xp, w_rel, b_rel, w_root, bn_g, bn_b, h1):
    srel = a0[...] + a1[...]
    cnt = jnp.maximum(srel[:, 4:5], 1.0)
    mean = srel[:, 0:4] / cnt
    h = (jnp.dot(mean, w_rel[...].T, preferred_element_type=jnp.float32)
         + b_rel[...][None, :]
         + jnp.dot(xp[:, 0:4], w_root[...].T,
                   preferred_element_type=jnp.float32))
    h = h * (bn_g[...] * lax.rsqrt(17.0))[None, :] + bn_b[...][None, :]
    h1[...] = jnp.maximum(h, 0.0)


def _tc_mid_body(eps, a0, a1, c0, c1, hin, w_rel, b_rel, w_root, bn_g, bn_b,
                 hout):
    srel = a0[...] + a1[...]
    cnt = jnp.maximum(c0[:, 4:5] + c1[:, 4:5], 1.0)
    mean = srel / cnt
    h = (jnp.dot(mean, w_rel[...].T, preferred_element_type=jnp.float32)
         + b_rel[...][None, :]
         + jnp.dot(hin[...], w_root[...].T,
                   preferred_element_type=jnp.float32))
    h = (h * (bn_g[...] * lax.rsqrt(1.0 + eps))[None, :]
         + bn_b[...][None, :])
    hout[...] = jnp.maximum(h, 0.0)


_WROWS = R // 128
_WBLK = 8
_WSTEPS = _WROWS // _WBLK


_BN = _WBLK * 128


def _tc4_body(a30, a31, c10, c11, h2b, batch_b, w_rel, b_rel, w_root, wg1,
              bg1, wg2, bg2, wfc1, bfc1, wfc4, bfc4, out, h3, gate_s, e_s,
              m_s, den_s, pooled_s):
    p = pl.program_id(0)
    i = pl.program_id(1)
    wsl = pl.ds(i * _WBLK, _WBLK)
    nsl = pl.ds(i * _BN, _BN)

    @pl.when(p == 0)
    def _phase0():
        cnt = jnp.maximum(c10[:, 4:5] + c11[:, 4:5], 1.0)
        mean = (a30[...] + a31[...]) / cnt
        h = (jnp.dot(mean, w_rel[...].T, preferred_element_type=jnp.float32)
             + b_rel[...][None, :]
             + jnp.dot(h2b[...], w_root[...].T,
                       preferred_element_type=jnp.float32))
        h = jnp.maximum(h, 0.0)
        h3[nsl, :] = h
        z1 = jnp.maximum(
            jnp.dot(h, wg1[...].T, preferred_element_type=jnp.float32)
            + bg1[...][None, :], 0.0)
        g = (jnp.sum(z1 * wg2[...], axis=1, keepdims=True)
             + bg2[...].reshape(1, 1))
        gate_s[wsl, :] = g.reshape(_WBLK, 128)

    bb = batch_b[...]
    iota3 = lax.broadcasted_iota(jnp.int32, (_WBLK, 128, G), 2)
    mask3 = bb[:, :, None] == iota3

    @pl.when(p == 1)
    def _phase_max():
        gb = gate_s[wsl, :]
        masked = jnp.where(mask3, gb[:, :, None], -1e30)
        mnew = jnp.max(masked, axis=(0, 1))[None, :]
        prev = jnp.where(i == 0, jnp.full((1, G), -1e30, jnp.float32),
                         m_s[...])
        m_s[...] = jnp.maximum(prev, mnew)

    @pl.when(p == 2)
    def _phase_den():
        @pl.when(i == 0)
        def _fix():
            mm = m_s[...]
            m_s[...] = jnp.where(mm > -0.9e30, mm, 0.0)
        gb = gate_s[wsl, :]
        m = m_s[...].reshape(G)
        m_pn = jnp.sum(jnp.where(mask3, m[None, None, :], 0.0), axis=2)
        e = jnp.where(bb < G, jnp.exp(gb - m_pn), 0.0)
        e_s[wsl, :] = e
        dnew = jnp.sum(jnp.where(mask3, e[:, :, None], 0.0),
                       axis=(0, 1))[None, :]
        den_s[...] = jnp.where(i == 0, 0.0, den_s[...]) + dnew

    @pl.when(p == 3)
    def _phase_pool():
        @pl.when(i == 0)
        def _inv():
            den_s[...] = 1.0 / jnp.maximum(den_s[...], 1e-16)
        e = e_s[wsl, :]
        invd = den_s[...].reshape(G)
        invd_pn = jnp.sum(jnp.where(mask3, invd[None, None, :], 0.0),
                          axis=2)
        alpha = e * invd_pn
        gi = lax.broadcasted_iota(jnp.int32, (G, 128), 0)
        acc = jnp.where(i == 0, 0.0, pooled_s[...])
        for j in range(_WBLK):
            onehot_t = gi == bb[j:j + 1, :]
            w = jnp.where(onehot_t, alpha[j:j + 1, :], 0.0)
            h3r = h3[pl.ds(i * _BN + j * 128, 128), :]
            acc = acc + jnp.dot(w, h3r, preferred_element_type=jnp.float32)
        pooled_s[...] = acc

        @pl.when(i == _WSTEPS - 1)
        def _final():
            z1 = jnp.maximum(
                jnp.dot(pooled_s[...], wfc1[...].T,
                        preferred_element_type=jnp.float32)
                + bfc1[...][None, :], 0.0)
            z = (jnp.dot(z1, wfc4[...].T,
                         preferred_element_type=jnp.float32)
                 + bfc4[...][None, :])
            zm = jnp.max(z, axis=1, keepdims=True)
            zz = z - zm
            out[...] = zz - jnp.log(jnp.sum(jnp.exp(zz), axis=1,
                                            keepdims=True))


def kernel(x, edge_weight, edge_attr, edge_index, batch, W1_rel, b1_rel,
           W1_root, bn1_g, bn1_b, W2_rel, b2_rel, W2_root, bn2_g, bn2_b,
           W3_rel, b3_rel, W3_root, Wg1, bg1, Wg2, bg2, Wfc1, bfc1, Wfc4,
           bfc4):
    f32 = jnp.float32
    pad_e = EP - E
    src = jnp.concatenate([edge_index[0], jnp.zeros((pad_e,), jnp.int32)])
    dst = jnp.concatenate([edge_index[1],
                           jnp.full((pad_e,), N, jnp.int32)])
    ew = jnp.concatenate([edge_weight, jnp.zeros((pad_e,), f32)])
    srcT = src.reshape(NC, NS, NBLK, CPB, CHUNK)
    dstT = dst.reshape(NC, NS, NBLK, CPB, CHUNK)
    ewT = ew.reshape(NC, NS, NBLK, CPB * CHUNK)

    xp = jnp.concatenate(
        [x, jnp.ones((N, 1), f32), jnp.zeros((N, 11), f32)], axis=1)
    xp = jnp.concatenate([xp, jnp.zeros((R - N, 16), f32)], axis=0)

    z16 = jnp.zeros((R, 16), f32)
    z32 = jnp.zeros((R, 32), f32)

    agg1 = _agg1(xp, srcT, dstT, ewT, z16)

    h1 = pl.pallas_call(
        _tc1_body,
        grid=(_TCGRID,),
        in_specs=[_rows(16), _rows2(16, _TCGRID), _rows(16),
                  _full((16, 4)), _full((16,)), _full((16, 4)),
                  _full((16,)), _full((16,))],
        out_specs=_rows(16),
        out_shape=jax.ShapeDtypeStruct((R, 16), f32),
    )(agg1, agg1, xp, W1_rel, b1_rel, W1_root, bn1_g, bn1_b)

    agg2 = _agg2(h1, srcT, dstT, ewT, z16)

    h2 = pl.pallas_call(
        functools.partial(_tc_mid_body, 32.0),
        grid=(_TCGRID,),
        in_specs=[_rows(16), _rows2(16, _TCGRID),
                  _rows(16), _rows2(16, _TCGRID), _rows(16),
                  _full((32, 16)), _full((32,)), _full((32, 16)),
                  _full((32,)), _full((32,))],
        out_specs=_rows(32),
        out_shape=jax.ShapeDtypeStruct((R, 32), f32),
    )(agg2, agg2, agg1, agg1, h1, W2_rel, b2_rel, W2_root, bn2_g, bn2_b)

    agg3 = _agg3(h2, srcT, dstT, ewT, z32)

    batch_w = jnp.concatenate(
        [batch, jnp.full((R - N,), G, jnp.int32)]).reshape(_WROWS, 128)

    def _p0(off):
        return lambda p, i: (off + jnp.where(p == 0, i, 0), 0)

    out = pl.pallas_call(
        _tc4_body,
        grid=(4, _WSTEPS),
        in_specs=[pl.BlockSpec((_BN, 32), _p0(0)),
                  pl.BlockSpec((_BN, 32), _p0(_WSTEPS)),
                  pl.BlockSpec((_BN, 16), _p0(0)),
                  pl.BlockSpec((_BN, 16), _p0(_WSTEPS)),
                  pl.BlockSpec((_BN, 32), _p0(0)),
                  pl.BlockSpec((_WBLK, 128), lambda p, i: (i, 0)),
                  _full((64, 32)), _full((64,)), _full((64, 32)),
                  _full((32, 64)), _full((32,)), _full((1, 32)),
                  _full((1,)),
                  _full((32, 64)), _full((32,)), _full((2, 32)),
                  _full((2,))],
        out_specs=_full((G, 2)),
        out_shape=jax.ShapeDtypeStruct((G, 2), f32),
        scratch_shapes=[pltpu.VMEM((R, 64), f32),
                        pltpu.VMEM((_WROWS, 128), f32),
                        pltpu.VMEM((_WROWS, 128), f32),
                        pltpu.VMEM((1, 128), f32),
                        pltpu.VMEM((1, 128), f32),
                        pltpu.VMEM((G, 64), f32)],
    )(agg3, agg3, agg1, agg1, h2, batch_w, W3_rel, b3_rel, W3_root, Wg1,
      bg1, Wg2, bg2, Wfc1, bfc1, Wfc4, bfc4)
    return out


# speedup vs baseline: 1.0252x; 1.0252x over previous
import functools

import jax
import jax.numpy as jnp
from jax import lax
from jax.experimental import pallas as pl
from jax.experimental.pallas import tpu as pltpu
from jax.experimental.pallas import tpu_sc as plsc

N = 50000
E = 800000
G = 128

NC = 2
NS = 16
NW = NC * NS
CHUNK = 128
CPB = 14
NBLK = 14
CPW = CPB * NBLK
EPW = CPW * CHUNK
EP = EPW * NW
RPT = 1568
R = RPT * NW
OUTC = 224


def _sc_aggregate(F, scale_all):
    mesh = plsc.VectorSubcoreMesh(core_axis_name="c", subcore_axis_name="s")

    @functools.partial(
        pl.kernel,
        mesh=mesh,
        out_type=jax.ShapeDtypeStruct((NC * R, F), jnp.float32),
        compiler_params=pltpu.CompilerParams(use_tc_tiling_on_sc=False),
        scratch_types=[
            pltpu.VMEM((CPB, CHUNK), jnp.int32),
            pltpu.VMEM((CPB, CHUNK), jnp.int32),
            pltpu.VMEM((CPB * CHUNK,), jnp.float32),
            pltpu.VMEM((CHUNK, F), jnp.float32),
            pltpu.VMEM((CHUNK, F), jnp.float32),
            pltpu.VMEM_SHARED((R, F), jnp.float32),
            pltpu.SemaphoreType.DMA,
            pltpu.SemaphoreType.DMA,
            pltpu.SemaphoreType.DMA,
            pltpu.SemaphoreType.DMA,
        ],
    )
    def agg(table, srcT, dstT, ewT, zrows, out, src_v, dst_v, ew_v, rows_v,
            rows_w, acc_sh, sem, sem2, sem3, sem4):
        c = lax.axis_index("c")
        s = lax.axis_index("s")
        row0 = s * RPT

        pltpu.sync_copy(zrows.at[pl.ds(row0, RPT)],
                        acc_sh.at[pl.ds(row0, RPT)])
        plsc.subcore_barrier()

        lane4 = lax.iota(jnp.int32, 16) < 4

        def scale(buf, ch):
            for k in range(CHUNK // 16):
                wvec = ew_v[pl.ds(ch * CHUNK + 16 * k, 16)]
                for jj in range(16):
                    j = 16 * k + jj
                    w = wvec[jj]
                    if scale_all:
                        for h in range(F // 16):
                            sl = pl.ds(16 * h, 16)
                            buf[j, sl] = buf[j, sl] * w
                    else:
                        sl = pl.ds(0, 16)
                        m = jnp.where(lane4, w, 1.0)
                        buf[j, sl] = buf[j, sl] * m

        def block(b, _):
            pltpu.sync_copy(srcT.at[c, s, b], src_v)
            pltpu.sync_copy(dstT.at[c, s, b], dst_v)
            pltpu.sync_copy(ewT.at[c, s, b], ew_v)
            pltpu.async_copy(table.at[src_v.at[0]], rows_v, sem)

            def pair(p, _):
                ch0 = 2 * p
                pltpu.make_async_copy(table.at[src_v.at[ch0]], rows_v,
                                      sem).wait()

                @pl.when(p > 0)
                def _drain_w():
                    pltpu.make_async_copy(rows_w, acc_sh.at[dst_v.at[ch0]],
                                          sem4).wait()
                pltpu.async_copy(table.at[src_v.at[ch0 + 1]], rows_w, sem2)
                scale(rows_v, ch0)
                pltpu.async_copy(rows_v, acc_sh.at[dst_v.at[ch0]], sem3,
                                 add=True)
                pltpu.make_async_copy(table.at[src_v.at[ch0 + 1]], rows_w,
                                      sem2).wait()
                pltpu.make_async_copy(rows_v, acc_sh.at[dst_v.at[ch0]],
                                      sem3).wait()

                @pl.when(p < CPB // 2 - 1)
                def _start_next():
                    pltpu.async_copy(table.at[src_v.at[ch0 + 2]], rows_v,
                                     sem)
                scale(rows_w, ch0 + 1)
                pltpu.async_copy(rows_w, acc_sh.at[dst_v.at[ch0 + 1]], sem4,
                                 add=True)
                return _
            lax.fori_loop(0, CPB // 2, pair, 0)
            pltpu.make_async_copy(rows_w, acc_sh.at[dst_v.at[CPB - 1]],
                                  sem4).wait()
            return _
        lax.fori_loop(0, NBLK, block, 0)
        plsc.subcore_barrier()

        pltpu.sync_copy(acc_sh.at[pl.ds(row0, RPT)],
                        out.at[pl.ds(c * R + row0, RPT)])

    return agg


_agg1 = _sc_aggregate(16, False)
_agg2 = _sc_aggregate(16, True)
_agg3 = _sc_aggregate(32, True)


BLK = RPT
_TCGRID = R // BLK


def _full(shape):
    return pl.BlockSpec(shape, lambda *_: tuple(0 for _ in shape))


def _rows(w):
    return pl.BlockSpec((BLK, w), lambda i: (i, 0))


def _rows2(w, off):
    return pl.BlockSpec((BLK, w), lambda i: (off + i, 0))


def _tc1_body(a0, a1, xp, w_rel, b_rel, w_root, bn_g, bn_b, h1):
    srel = a0[...] + a1[...]
    cnt = jnp.maximum(srel[:, 4:5], 1.0)
    mean = srel[:, 0:4] / cnt
    h = (jnp.dot(mean, w_rel[...].T, preferred_element_type=jnp.float32)
         + b_rel[...][None, :]
         + jnp.dot(xp[:, 0:4], w_root[...].T,
                   preferred_element_type=jnp.float32))
    h = h * (bn_g[...] * lax.rsqrt(17.0))[None, :] + bn_b[...][None, :]
    h1[...] = jnp.maximum(h, 0.0)


def _tc_mid_body(eps, a0, a1, c0, c1, hin, w_rel, b_rel, w_root, bn_g, bn_b,
                 hout):
    srel = a0[...] + a1[...]
    cnt = jnp.maximum(c0[:, 4:5] + c1[:, 4:5], 1.0)
    mean = srel / cnt
    h = (jnp.dot(mean, w_rel[...].T, preferred_element_type=jnp.float32)
         + b_rel[...][None, :]
         + jnp.dot(hin[...], w_root[...].T,
                   preferred_element_type=jnp.float32))
    h = (h * (bn_g[...] * lax.rsqrt(1.0 + eps))[None, :]
         + bn_b[...][None, :])
    hout[...] = jnp.maximum(h, 0.0)


_WROWS = R // 128
_WBLK = 8
_WSTEPS = _WROWS // _WBLK


def _tc3_body(a0, a1, c0, c1, hin, w_rel, b_rel, w_root, wg1, bg1, wg2,
              bg2, h3, gate):
    srel = a0[...] + a1[...]
    cnt = jnp.maximum(c0[:, 4:5] + c1[:, 4:5], 1.0)
    mean = srel / cnt
    h = (jnp.dot(mean, w_rel[...].T, preferred_element_type=jnp.float32)
         + b_rel[...][None, :]
         + jnp.dot(hin[...], w_root[...].T,
                   preferred_element_type=jnp.float32))
    h = jnp.maximum(h, 0.0)
    h3[...] = h
    z1 = jnp.maximum(
        jnp.dot(h, wg1[...].T, preferred_element_type=jnp.float32)
        + bg1[...][None, :], 0.0)
    gate[...] = (jnp.sum(z1 * wg2[...], axis=1, keepdims=True)
                 + bg2[...].reshape(1, 1))


def _tc4_body(gate_w, batch_w, h3, wfc1, bfc1, wfc4, bfc4, out, e_w):
    iota3 = lax.broadcasted_iota(jnp.int32, (_WBLK, 128, G), 2)

    def blk(i):
        gb = gate_w[pl.ds(i * _WBLK, _WBLK), :]
        bb = batch_w[pl.ds(i * _WBLK, _WBLK), :]
        return gb, bb, bb[:, :, None] == iota3

    def phase_a(i, m):
        gb, bb, mask3 = blk(i)
        masked = jnp.where(mask3, gb[:, :, None], -1e30)
        return jnp.maximum(m, jnp.max(masked, axis=(0, 1)))

    m = lax.fori_loop(0, _WSTEPS, phase_a,
                      jnp.full((G,), -1e30, jnp.float32))
    m = jnp.where(m > -0.9e30, m, 0.0)

    def phase_b(i, den):
        gb, bb, mask3 = blk(i)
        m_pn = jnp.sum(jnp.where(mask3, m[None, None, :], 0.0), axis=2)
        e = jnp.where(bb < G, jnp.exp(gb - m_pn), 0.0)
        e_w[pl.ds(i * _WBLK, _WBLK), :] = e
        return den + jnp.sum(jnp.where(mask3, e[:, :, None], 0.0),
                             axis=(0, 1))

    den = lax.fori_loop(0, _WSTEPS, phase_b, jnp.zeros((G,), jnp.float32))
    invd = 1.0 / jnp.maximum(den, 1e-16)

    def phase_c(i, pooled):
        _, bb, mask3 = blk(i)
        e = e_w[pl.ds(i * _WBLK, _WBLK), :]
        invd_pn = jnp.sum(jnp.where(mask3, invd[None, None, :], 0.0),
                          axis=2)
        alpha = e * invd_pn
        gi = lax.broadcasted_iota(jnp.int32, (G, 128), 0)
        for j in range(_WBLK):
            onehot_t = gi == bb[j:j + 1, :]
            w = jnp.where(onehot_t, alpha[j:j + 1, :], 0.0)
            h3r = h3[pl.ds(i * _WBLK * 128 + j * 128, 128), :]
            pooled = pooled + jnp.dot(w, h3r,
                                      preferred_element_type=jnp.float32)
        return pooled

    pooled = lax.fori_loop(0, _WSTEPS, phase_c,
                           jnp.zeros((G, 64), jnp.float32))
    z1 = jnp.maximum(
        jnp.dot(pooled, wfc1[...].T, preferred_element_type=jnp.float32)
        + bfc1[...][None, :], 0.0)
    z = (jnp.dot(z1, wfc4[...].T, preferred_element_type=jnp.float32)
         + bfc4[...][None, :])
    zm = jnp.max(z, axis=1, keepdims=True)
    zz = z - zm
    out[...] = zz - jnp.log(jnp.sum(jnp.exp(zz), axis=1, keepdims=True))


def kernel(x, edge_weight, edge_attr, edge_index, batch, W1_rel, b1_rel,
           W1_root, bn1_g, bn1_b, W2_rel, b2_rel, W2_root, bn2_g, bn2_b,
           W3_rel, b3_rel, W3_root, Wg1, bg1, Wg2, bg2, Wfc1, bfc1, Wfc4,
           bfc4):
    f32 = jnp.float32
    pad_e = EP - E
    src = jnp.concatenate([edge_index[0], jnp.zeros((pad_e,), jnp.int32)])
    dst = jnp.concatenate([edge_index[1],
                           jnp.full((pad_e,), N, jnp.int32)])
    ew = jnp.concatenate([edge_weight, jnp.zeros((pad_e,), f32)])
    srcT = src.reshape(NC, NS, NBLK, CPB, CHUNK)
    dstT = dst.reshape(NC, NS, NBLK, CPB, CHUNK)
    ewT = ew.reshape(NC, NS, NBLK, CPB * CHUNK)

    xp = jnp.concatenate(
        [x, jnp.ones((N, 1), f32), jnp.zeros((N, 11), f32)], axis=1)
    xp = jnp.concatenate([xp, jnp.zeros((R - N, 16), f32)], axis=0)

    z16 = jnp.zeros((R, 16), f32)
    z32 = jnp.zeros((R, 32), f32)

    agg1 = _agg1(xp, srcT, dstT, ewT, z16)

    h1 = pl.pallas_call(
        _tc1_body,
        grid=(_TCGRID,),
        in_specs=[_rows(16), _rows2(16, _TCGRID), _rows(16),
                  _full((16, 4)), _full((16,)), _full((16, 4)),
                  _full((16,)), _full((16,))],
        out_specs=_rows(16),
        out_shape=jax.ShapeDtypeStruct((R, 16), f32),
    )(agg1, agg1, xp, W1_rel, b1_rel, W1_root, bn1_g, bn1_b)

    agg2 = _agg2(h1, srcT, dstT, ewT, z16)

    h2 = pl.pallas_call(
        functools.partial(_tc_mid_body, 32.0),
        grid=(_TCGRID,),
        in_specs=[_rows(16), _rows2(16, _TCGRID),
                  _rows(16), _rows2(16, _TCGRID), _rows(16),
                  _full((32, 16)), _full((32,)), _full((32, 16)),
                  _full((32,)), _full((32,))],
        out_specs=_rows(32),
        out_shape=jax.ShapeDtypeStruct((R, 32), f32),
    )(agg2, agg2, agg1, agg1, h1, W2_rel, b2_rel, W2_root, bn2_g, bn2_b)

    agg3 = _agg3(h2, srcT, dstT, ewT, z32)

    h3, gate = pl.pallas_call(
        _tc3_body,
        grid=(_TCGRID,),
        in_specs=[_rows(32), _rows2(32, _TCGRID),
                  _rows(16), _rows2(16, _TCGRID), _rows(32),
                  _full((64, 32)), _full((64,)), _full((64, 32)),
                  _full((32, 64)), _full((32,)), _full((1, 32)),
                  _full((1,))],
        out_specs=[_rows(64), _rows(1)],
        out_shape=[jax.ShapeDtypeStruct((R, 64), f32),
                   jax.ShapeDtypeStruct((R, 1), f32)],
    )(agg3, agg3, agg1, agg1, h2, W3_rel, b3_rel, W3_root, Wg1, bg1, Wg2,
      bg2)

    gate_w = gate.reshape(_WROWS, 128)
    batch_w = jnp.concatenate(
        [batch, jnp.full((R - N,), G, jnp.int32)]).reshape(_WROWS, 128)

    out = pl.pallas_call(
        _tc4_body,
        in_specs=[_full((_WROWS, 128)), _full((_WROWS, 128)),
                  _full((R, 64)), _full((32, 64)), _full((32,)),
                  _full((2, 32)), _full((2,))],
        out_specs=_full((G, 2)),
        out_shape=jax.ShapeDtypeStruct((G, 2), f32),
        scratch_shapes=[pltpu.VMEM((_WROWS, 128), f32)],
    )(gate_w, batch_w, h3, Wfc1, bfc1, Wfc4, bfc4)
    return out


# speedup vs baseline: 1.0622x; 1.0360x over previous
import functools

import jax
import jax.numpy as jnp
from jax import lax
from jax.experimental import pallas as pl
from jax.experimental.pallas import tpu as pltpu
from jax.experimental.pallas import tpu_sc as plsc

N = 50000
E = 800000
G = 128

NC = 2
NS = 16
NW = NC * NS
CHUNK = 128
CPB = 28
NBLK = 7
CPW = CPB * NBLK
EPW = CPW * CHUNK
EP = EPW * NW
RPT = 1568
R = RPT * NW
OUTC = 224


def _sc_aggregate(F, scale_all):
    mesh = plsc.VectorSubcoreMesh(core_axis_name="c", subcore_axis_name="s")

    @functools.partial(
        pl.kernel,
        mesh=mesh,
        out_type=jax.ShapeDtypeStruct((NC * R, F), jnp.float32),
        compiler_params=pltpu.CompilerParams(use_tc_tiling_on_sc=False),
        scratch_types=[
            pltpu.VMEM((CPB, CHUNK), jnp.int32),
            pltpu.VMEM((CPB, CHUNK), jnp.int32),
            pltpu.VMEM((CPB * CHUNK,), jnp.float32),
            pltpu.VMEM((CHUNK, F), jnp.float32),
            pltpu.VMEM((CHUNK, F), jnp.float32),
            pltpu.VMEM_SHARED((R, F), jnp.float32),
            pltpu.SemaphoreType.DMA,
            pltpu.SemaphoreType.DMA,
            pltpu.SemaphoreType.DMA,
            pltpu.SemaphoreType.DMA,
        ],
    )
    def agg(table, srcT, dstT, ewT, zrows, out, src_v, dst_v, ew_v, rows_v,
            rows_w, acc_sh, sem, sem2, sem3, sem4):
        c = lax.axis_index("c")
        s = lax.axis_index("s")
        row0 = s * RPT

        pltpu.sync_copy(zrows.at[pl.ds(row0, RPT)],
                        acc_sh.at[pl.ds(row0, RPT)])
        plsc.subcore_barrier()

        lane4 = lax.iota(jnp.int32, 16) < 4

        def scale(buf, ch):
            for k in range(CHUNK // 16):
                wvec = ew_v[pl.ds(ch * CHUNK + 16 * k, 16)]
                for jj in range(16):
                    j = 16 * k + jj
                    w = wvec[jj]
                    if scale_all:
                        for h in range(F // 16):
                            sl = pl.ds(16 * h, 16)
                            buf[j, sl] = buf[j, sl] * w
                    else:
                        sl = pl.ds(0, 16)
                        m = jnp.where(lane4, w, 1.0)
                        buf[j, sl] = buf[j, sl] * m

        def block(b, _):
            pltpu.sync_copy(srcT.at[c, s, b], src_v)
            pltpu.sync_copy(dstT.at[c, s, b], dst_v)
            pltpu.sync_copy(ewT.at[c, s, b], ew_v)
            pltpu.async_copy(table.at[src_v.at[0]], rows_v, sem)

            def pair(p, _):
                ch0 = 2 * p
                pltpu.make_async_copy(table.at[src_v.at[ch0]], rows_v,
                                      sem).wait()

                @pl.when(p > 0)
                def _drain_w():
                    pltpu.make_async_copy(rows_w, acc_sh.at[dst_v.at[ch0]],
                                          sem4).wait()
                pltpu.async_copy(table.at[src_v.at[ch0 + 1]], rows_w, sem2)
                scale(rows_v, ch0)
                pltpu.async_copy(rows_v, acc_sh.at[dst_v.at[ch0]], sem3,
                                 add=True)
                pltpu.make_async_copy(table.at[src_v.at[ch0 + 1]], rows_w,
                                      sem2).wait()
                pltpu.make_async_copy(rows_v, acc_sh.at[dst_v.at[ch0]],
                                      sem3).wait()

                @pl.when(p < CPB // 2 - 1)
                def _start_next():
                    pltpu.async_copy(table.at[src_v.at[ch0 + 2]], rows_v,
                                     sem)
                scale(rows_w, ch0 + 1)
                pltpu.async_copy(rows_w, acc_sh.at[dst_v.at[ch0 + 1]], sem4,
                                 add=True)
                return _
            lax.fori_loop(0, CPB // 2, pair, 0)
            pltpu.make_async_copy(rows_w, acc_sh.at[dst_v.at[CPB - 1]],
                                  sem4).wait()
            return _
        lax.fori_loop(0, NBLK, block, 0)
        plsc.subcore_barrier()

        pltpu.sync_copy(acc_sh.at[pl.ds(row0, RPT)],
                        out.at[pl.ds(c * R + row0, RPT)])

    return agg


_agg1 = _sc_aggregate(16, False)
_agg2 = _sc_aggregate(16, True)
_agg3 = _sc_aggregate(32, True)


BLK = RPT
_TCGRID = R // BLK


def _full(shape):
    return pl.BlockSpec(shape, lambda *_: tuple(0 for _ in shape))


def _rows(w):
    return pl.BlockSpec((BLK, w), lambda i: (i, 0))


def _rows2(w, off):
    return pl.BlockSpec((BLK, w), lambda i: (off + i, 0))


def _tc1_body(a0, a1, xp, w_rel, b_rel, w_root, bn_g, bn_b, h1):
    srel = a0[...] + a1[...]
    cnt = jnp.maximum(srel[:, 4:5], 1.0)
    mean = srel[:, 0:4] / cnt
    h = (jnp.dot(mean, w_rel[...].T, preferred_element_type=jnp.float32)
         + b_rel[...][None, :]
         + jnp.dot(xp[:, 0:4], w_root[...].T,
                   preferred_element_type=jnp.float32))
    h = h * (bn_g[...] * lax.rsqrt(17.0))[None, :] + bn_b[...][None, :]
    h1[...] = jnp.maximum(h, 0.0)


def _tc_mid_body(eps, a0, a1, c0, c1, hin, w_rel, b_rel, w_root, bn_g, bn_b,
                 hout):
    srel = a0[...] + a1[...]
    cnt = jnp.maximum(c0[:, 4:5] + c1[:, 4:5], 1.0)
    mean = srel / cnt
    h = (jnp.dot(mean, w_rel[...].T, preferred_element_type=jnp.float32)
         + b_rel[...][None, :]
         + jnp.dot(hin[...], w_root[...].T,
                   preferred_element_type=jnp.float32))
    h = (h * (bn_g[...] * lax.rsqrt(1.0 + eps))[None, :]
         + bn_b[...][None, :])
    hout[...] = jnp.maximum(h, 0.0)


_WROWS = R // 128
_WBLK = 8
_WSTEPS = _WROWS // _WBLK


def _tc3_body(a0, a1, c0, c1, hin, w_rel, b_rel, w_root, wg1, bg1, wg2,
              bg2, h3, gate):
    srel = a0[...] + a1[...]
    cnt = jnp.maximum(c0[:, 4:5] + c1[:, 4:5], 1.0)
    mean = srel / cnt
    h = (jnp.dot(mean, w_rel[...].T, preferred_element_type=jnp.float32)
         + b_rel[...][None, :]
         + jnp.dot(hin[...], w_root[...].T,
                   preferred_element_type=jnp.float32))
    h = jnp.maximum(h, 0.0)
    h3[...] = h
    z1 = jnp.maximum(
        jnp.dot(h, wg1[...].T, preferred_element_type=jnp.float32)
        + bg1[...][None, :], 0.0)
    gate[...] = (jnp.sum(z1 * wg2[...], axis=1, keepdims=True)
                 + bg2[...].reshape(1, 1))


def _tc4_body(gate_w, batch_w, h3, wfc1, bfc1, wfc4, bfc4, out, e_w):
    iota3 = lax.broadcasted_iota(jnp.int32, (_WBLK, 128, G), 2)

    def blk(i):
        gb = gate_w[pl.ds(i * _WBLK, _WBLK), :]
        bb = batch_w[pl.ds(i * _WBLK, _WBLK), :]
        return gb, bb, bb[:, :, None] == iota3

    def phase_a(i, m):
        gb, bb, mask3 = blk(i)
        masked = jnp.where(mask3, gb[:, :, None], -1e30)
        return jnp.maximum(m, jnp.max(masked, axis=(0, 1)))

    m = lax.fori_loop(0, _WSTEPS, phase_a,
                      jnp.full((G,), -1e30, jnp.float32))
    m = jnp.where(m > -0.9e30, m, 0.0)

    def phase_b(i, den):
        gb, bb, mask3 = blk(i)
        m_pn = jnp.sum(jnp.where(mask3, m[None, None, :], 0.0), axis=2)
        e = jnp.where(bb < G, jnp.exp(gb - m_pn), 0.0)
        e_w[pl.ds(i * _WBLK, _WBLK), :] = e
        return den + jnp.sum(jnp.where(mask3, e[:, :, None], 0.0),
                             axis=(0, 1))

    den = lax.fori_loop(0, _WSTEPS, phase_b, jnp.zeros((G,), jnp.float32))
    invd = 1.0 / jnp.maximum(den, 1e-16)

    def phase_c(i, pooled):
        _, bb, mask3 = blk(i)
        e = e_w[pl.ds(i * _WBLK, _WBLK), :]
        invd_pn = jnp.sum(jnp.where(mask3, invd[None, None, :], 0.0),
                          axis=2)
        alpha = e * invd_pn
        gi = lax.broadcasted_iota(jnp.int32, (G, 128), 0)
        for j in range(_WBLK):
            onehot_t = gi == bb[j:j + 1, :]
            w = jnp.where(onehot_t, alpha[j:j + 1, :], 0.0)
            h3r = h3[pl.ds(i * _WBLK * 128 + j * 128, 128), :]
            pooled = pooled + jnp.dot(w, h3r,
                                      preferred_element_type=jnp.float32)
        return pooled

    pooled = lax.fori_loop(0, _WSTEPS, phase_c,
                           jnp.zeros((G, 64), jnp.float32))
    z1 = jnp.maximum(
        jnp.dot(pooled, wfc1[...].T, preferred_element_type=jnp.float32)
        + bfc1[...][None, :], 0.0)
    z = (jnp.dot(z1, wfc4[...].T, preferred_element_type=jnp.float32)
         + bfc4[...][None, :])
    zm = jnp.max(z, axis=1, keepdims=True)
    zz = z - zm
    out[...] = zz - jnp.log(jnp.sum(jnp.exp(zz), axis=1, keepdims=True))


def kernel(x, edge_weight, edge_attr, edge_index, batch, W1_rel, b1_rel,
           W1_root, bn1_g, bn1_b, W2_rel, b2_rel, W2_root, bn2_g, bn2_b,
           W3_rel, b3_rel, W3_root, Wg1, bg1, Wg2, bg2, Wfc1, bfc1, Wfc4,
           bfc4):
    f32 = jnp.float32
    pad_e = EP - E
    src = jnp.concatenate([edge_index[0], jnp.zeros((pad_e,), jnp.int32)])
    dst = jnp.concatenate([edge_index[1],
                           jnp.full((pad_e,), N, jnp.int32)])
    ew = jnp.concatenate([edge_weight, jnp.zeros((pad_e,), f32)])
    srcT = src.reshape(NC, NS, NBLK, CPB, CHUNK)
    dstT = dst.reshape(NC, NS, NBLK, CPB, CHUNK)
    ewT = ew.reshape(NC, NS, NBLK, CPB * CHUNK)

    xp = jnp.concatenate(
        [x, jnp.ones((N, 1), f32), jnp.zeros((N, 11), f32)], axis=1)
    xp = jnp.concatenate([xp, jnp.zeros((R - N, 16), f32)], axis=0)

    z16 = jnp.zeros((R, 16), f32)
    z32 = jnp.zeros((R, 32), f32)

    agg1 = _agg1(xp, srcT, dstT, ewT, z16)

    h1 = pl.pallas_call(
        _tc1_body,
        grid=(_TCGRID,),
        in_specs=[_rows(16), _rows2(16, _TCGRID), _rows(16),
                  _full((16, 4)), _full((16,)), _full((16, 4)),
                  _full((16,)), _full((16,))],
        out_specs=_rows(16),
        out_shape=jax.ShapeDtypeStruct((R, 16), f32),
    )(agg1, agg1, xp, W1_rel, b1_rel, W1_root, bn1_g, bn1_b)

    agg2 = _agg2(h1, srcT, dstT, ewT, z16)

    h2 = pl.pallas_call(
        functools.partial(_tc_mid_body, 32.0),
        grid=(_TCGRID,),
        in_specs=[_rows(16), _rows2(16, _TCGRID),
                  _rows(16), _rows2(16, _TCGRID), _rows(16),
                  _full((32, 16)), _full((32,)), _full((32, 16)),
                  _full((32,)), _full((32,))],
        out_specs=_rows(32),
        out_shape=jax.ShapeDtypeStruct((R, 32), f32),
    )(agg2, agg2, agg1, agg1, h1, W2_rel, b2_rel, W2_root, bn2_g, bn2_b)

    agg3 = _agg3(h2, srcT, dstT, ewT, z32)

    h3, gate = pl.pallas_call(
        _tc3_body,
        grid=(_TCGRID,),
        in_specs=[_rows(32), _rows2(32, _TCGRID),
                  _rows(16), _rows2(16, _TCGRID), _rows(32),
                  _full((64, 32)), _full((64,)), _full((64, 32)),
                  _full((32, 64)), _full((32,)), _full((1, 32)),
                  _full((1,))],
        out_specs=[_rows(64), _rows(1)],
        out_shape=[jax.ShapeDtypeStruct((R, 64), f32),
                   jax.ShapeDtypeStruct((R, 1), f32)],
    )(agg3, agg3, agg1, agg1, h2, W3_rel, b3_rel, W3_root, Wg1, bg1, Wg2,
      bg2)

    gate_w = gate.reshape(_WROWS, 128)
    batch_w = jnp.concatenate(
        [batch, jnp.full((R - N,), G, jnp.int32)]).reshape(_WROWS, 128)

    out = pl.pallas_call(
        _tc4_body,
        in_specs=[_full((_WROWS, 128)), _full((_WROWS, 128)),
                  _full((R, 64)), _full((32, 64)), _full((32,)),
                  _full((2, 32)), _full((2,))],
        out_specs=_full((G, 2)),
        out_shape=jax.ShapeDtypeStruct((G, 2), f32),
        scratch_shapes=[pltpu.VMEM((_WROWS, 128), f32)],
    )(gate_w, batch_w, h3, Wfc1, bfc1, Wfc4, bfc4)
    return out
